# baseline (speedup 1.0000x reference)
import functools

import jax
import jax.numpy as jnp
from jax import lax
from jax.experimental import pallas as pl
from jax.experimental.pallas import tpu as pltpu
from jax.experimental.pallas import tpu_sc as plsc

_N = 10000
_E = 320000
_F = 128
_H = 16
_O = 7

_NP = 10240
_NG = 8
_FPT = 4
_EB = 800
_NBLK0 = 78
_NBLK1 = 22
_GPB = _EB // 16
_G0 = _NBLK0 * _EB
_G1 = _NBLK1 * _EB


def _mm1_body(x_ref, w_ref, o_ref):
    hT = lax.dot_general(w_ref[:], x_ref[:], (((0,), (1,)), ((), ())),
                         preferred_element_type=jnp.float32)
    o_ref[:] = jnp.concatenate(
        [hT, jnp.zeros((_H, _NP - _N), jnp.float32)], axis=1)


def _mm1(x, W1):
    return pl.pallas_call(
        _mm1_body,
        out_shape=jax.ShapeDtypeStruct((_H, _NP), jnp.float32),
    )(x, W1)


def _mid_body(p_ref, b1_ref, w2_ref, o_ref):
    aggT = jnp.sum(p_ref[:], axis=0) + b1_ref[:]
    o_ref[:] = lax.dot_general(w2_ref[:], aggT, (((0,), (0,)), ((), ())),
                               preferred_element_type=jnp.float32)


def _mid(p1, b1_col, W2p):
    return pl.pallas_call(
        _mid_body,
        out_shape=jax.ShapeDtypeStruct((_H, _NP), jnp.float32),
    )(p1, b1_col, W2p)


def _fin_body(p_ref, b2_ref, eye_ref, o_ref):
    aggT = jnp.sum(p_ref[:], axis=0)
    agg = lax.dot_general(aggT, eye_ref[:], (((0,), (0,)), ((), ())),
                          preferred_element_type=jnp.float32)
    o_ref[:] = agg[:_N, :_O] + b2_ref[:]


def _fin(p2, b2_2d):
    return pl.pallas_call(
        _fin_body,
        out_shape=jax.ShapeDtypeStruct((_N, _O), jnp.float32),
    )(p2, b2_2d, jnp.eye(_H, dtype=jnp.float32))


def _sc_agg(hT_flat, ei, w):
    mesh = plsc.VectorSubcoreMesh(core_axis_name="c", subcore_axis_name="s")

    @functools.partial(
        pl.kernel,
        out_type=jax.ShapeDtypeStruct((_NG, _H, _NP), jnp.float32),
        mesh=mesh,
        scratch_types=[
            pltpu.VMEM((_FPT, _NP), jnp.float32),
            pltpu.VMEM((_NP,), jnp.float32),
            pltpu.VMEM((_NP,), jnp.float32),
            pltpu.VMEM((_NP,), jnp.float32),
            pltpu.VMEM((_NP,), jnp.float32),
            pltpu.VMEM((2, _EB), jnp.int32),
            pltpu.VMEM((2, _EB), jnp.int32),
            pltpu.VMEM((2, _EB), jnp.float32),
            pltpu.SemaphoreType.DMA,
            pltpu.SemaphoreType.DMA,
            pltpu.SemaphoreType.DMA,
        ],
        compiler_params=pltpu.CompilerParams(
            needs_layout_passes=False, use_tc_tiling_on_sc=False),
    )
    def k(hT_hbm, ei_hbm, w_hbm, out_hbm, hT_v, a0, a1, a2, a3,
          eb_src, eb_dst, eb_w, sem0, sem1, semh):
        accs = (a0, a1, a2, a3)
        cid = lax.axis_index("c")
        sid = lax.axis_index("s")
        wid = cid * 16 + sid
        grp = wid // _FPT
        fbase = (wid % _FPT) * _FPT

        hT_dma = pltpu.make_async_copy(
            hT_hbm.at[pl.ds(fbase, _FPT)], hT_v, semh)
        hT_dma.start()

        zero16 = jnp.zeros((16,), jnp.float32)

        @plsc.parallel_loop(0, _NP // 32, unroll=8)
        def zb(i):
            for fi in range(_FPT):
                for u in range(2):
                    accs[fi][pl.ds((i * 2 + u) * 16, 16)] = zero16

        ebase0 = jnp.where(cid == 0, grp * _G0,
                           4 * _G0 + (grp - 4) * _G1)

        def start(j, par, sem):
            e0 = pl.multiple_of(ebase0 + j * _EB, 8)
            pltpu.make_async_copy(
                ei_hbm.at[0, pl.ds(e0, _EB)], eb_src.at[par], sem).start()
            pltpu.make_async_copy(
                ei_hbm.at[1, pl.ds(e0, _EB)], eb_dst.at[par], sem).start()
            pltpu.make_async_copy(
                w_hbm.at[pl.ds(e0, _EB)], eb_w.at[par], sem).start()

        def wait(par, sem):
            pltpu.make_async_copy(
                ei_hbm.at[0, pl.ds(0, _EB)], eb_src.at[par], sem).wait()
            pltpu.make_async_copy(
                ei_hbm.at[1, pl.ds(0, _EB)], eb_dst.at[par], sem).wait()
            pltpu.make_async_copy(
                w_hbm.at[pl.ds(0, _EB)], eb_w.at[par], sem).wait()

        start(0, 0, sem0)
        hT_dma.wait()

        def process(par):
            sb = eb_src.at[par]
            db = eb_dst.at[par]
            wb = eb_w.at[par]

            @plsc.parallel_loop(0, _GPB, unroll=10)
            def body(g):
                o = g * 16
                src16 = sb[pl.ds(o, 16)]
                dst16 = db[pl.ds(o, 16)]
                wp = wb[pl.ds(o, 16)]
                for fi in range(_FPT):
                    vals = plsc.load_gather(hT_v.at[fi], [src16]) * wp
                    plsc.addupdate_scatter(accs[fi], [dst16], vals)

        def run(nblk):
            def blk(jj, carry):
                j0 = jj * 2
                wait(0, sem0)
                start(j0 + 1, 1, sem1)
                process(0)
                wait(1, sem1)

                @pl.when(j0 + 2 < nblk)
                def _():
                    start(j0 + 2, 0, sem0)
                process(1)
                return carry
            lax.fori_loop(0, nblk // 2, blk, 0)

        @pl.when(cid == 0)
        def _():
            run(_NBLK0)

        @pl.when(cid == 1)
        def _():
            run(_NBLK1)

        for fi in range(_FPT):
            pltpu.sync_copy(accs[fi], out_hbm.at[grp, fbase + fi])

    return k(hT_flat, ei, w)


def kernel(x, edge_index, edge_weight, W1, b1, W2, b2):
    W2p = jnp.pad(W2, ((0, 0), (0, _H - _O)))
    b1_col = b1.reshape(_H, 1)
    b2_2d = b2.reshape(1, _O)

    h1T = _mm1(x, W1)
    p1 = _sc_agg(h1T, edge_index, edge_weight)
    h2T = _mid(p1, b1_col, W2p)
    p2 = _sc_agg(h2T, edge_index, edge_weight)
    return _fin(p2, b2_2d)


# speedup vs baseline: 1.0579x; 1.0579x over previous
import functools

import jax
import jax.numpy as jnp
from jax import lax
from jax.experimental import pallas as pl
from jax.experimental.pallas import tpu as pltpu
from jax.experimental.pallas import tpu_sc as plsc

_N = 10000
_E = 320000
_F = 128
_H = 16
_O = 7

_NP = 10240
_NG = 8
_FPT = 4
_EB = 800
_NBLK0 = 70
_NBLK1 = 30
_GPB = _EB // 16
_G0 = _NBLK0 * _EB
_G1 = _NBLK1 * _EB


def _mm1_body(x_ref, w_ref, o_ref):
    hT = lax.dot_general(w_ref[:], x_ref[:], (((0,), (1,)), ((), ())),
                         preferred_element_type=jnp.float32)
    o_ref[:] = jnp.concatenate(
        [hT, jnp.zeros((_H, _NP - _N), jnp.float32)], axis=1)


def _mm1(x, W1):
    return pl.pallas_call(
        _mm1_body,
        out_shape=jax.ShapeDtypeStruct((_H, _NP), jnp.float32),
    )(x, W1)


def _mid_body(p_ref, b1_ref, w2_ref, o_ref):
    aggT = jnp.sum(p_ref[:], axis=0) + b1_ref[:]
    o_ref[:] = lax.dot_general(w2_ref[:], aggT, (((0,), (0,)), ((), ())),
                               preferred_element_type=jnp.float32)


def _mid(p1, b1_col, W2p):
    return pl.pallas_call(
        _mid_body,
        out_shape=jax.ShapeDtypeStruct((_H, _NP), jnp.float32),
    )(p1, b1_col, W2p)


def _fin_body(p_ref, b2_ref, eye_ref, o_ref):
    aggT = jnp.sum(p_ref[:], axis=0)
    agg = lax.dot_general(aggT, eye_ref[:], (((0,), (0,)), ((), ())),
                          preferred_element_type=jnp.float32)
    o_ref[:] = agg[:_N, :_O] + b2_ref[:]


def _fin(p2, b2_2d):
    return pl.pallas_call(
        _fin_body,
        out_shape=jax.ShapeDtypeStruct((_N, _O), jnp.float32),
    )(p2, b2_2d, jnp.eye(_H, dtype=jnp.float32))


def _sc_agg(hT_flat, ei, w):
    mesh = plsc.VectorSubcoreMesh(core_axis_name="c", subcore_axis_name="s")

    @functools.partial(
        pl.kernel,
        out_type=jax.ShapeDtypeStruct((_NG, _H, _NP), jnp.float32),
        mesh=mesh,
        scratch_types=[
            pltpu.VMEM((_FPT, _NP), jnp.float32),
            pltpu.VMEM((_NP,), jnp.float32),
            pltpu.VMEM((_NP,), jnp.float32),
            pltpu.VMEM((_NP,), jnp.float32),
            pltpu.VMEM((_NP,), jnp.float32),
            pltpu.VMEM((2, _EB), jnp.int32),
            pltpu.VMEM((2, _EB), jnp.int32),
            pltpu.VMEM((2, _EB), jnp.float32),
            pltpu.SemaphoreType.DMA,
            pltpu.SemaphoreType.DMA,
            pltpu.SemaphoreType.DMA,
        ],
        compiler_params=pltpu.CompilerParams(
            needs_layout_passes=False, use_tc_tiling_on_sc=False),
    )
    def k(hT_hbm, ei_hbm, w_hbm, out_hbm, hT_v, a0, a1, a2, a3,
          eb_src, eb_dst, eb_w, sem0, sem1, semh):
        accs = (a0, a1, a2, a3)
        cid = lax.axis_index("c")
        sid = lax.axis_index("s")
        wid = cid * 16 + sid
        grp = wid // _FPT
        fbase = (wid % _FPT) * _FPT

        hT_dma = pltpu.make_async_copy(
            hT_hbm.at[pl.ds(fbase, _FPT)], hT_v, semh)
        hT_dma.start()

        zero16 = jnp.zeros((16,), jnp.float32)

        @plsc.parallel_loop(0, _NP // 32, unroll=8)
        def zb(i):
            for fi in range(_FPT):
                for u in range(2):
                    accs[fi][pl.ds((i * 2 + u) * 16, 16)] = zero16

        ebase0 = jnp.where(cid == 0, grp * _G0,
                           4 * _G0 + (grp - 4) * _G1)

        def start(j, par, sem):
            e0 = pl.multiple_of(ebase0 + j * _EB, 8)
            pltpu.make_async_copy(
                ei_hbm.at[0, pl.ds(e0, _EB)], eb_src.at[par], sem).start()
            pltpu.make_async_copy(
                ei_hbm.at[1, pl.ds(e0, _EB)], eb_dst.at[par], sem).start()
            pltpu.make_async_copy(
                w_hbm.at[pl.ds(e0, _EB)], eb_w.at[par], sem).start()

        def wait(par, sem):
            pltpu.make_async_copy(
                ei_hbm.at[0, pl.ds(0, _EB)], eb_src.at[par], sem).wait()
            pltpu.make_async_copy(
                ei_hbm.at[1, pl.ds(0, _EB)], eb_dst.at[par], sem).wait()
            pltpu.make_async_copy(
                w_hbm.at[pl.ds(0, _EB)], eb_w.at[par], sem).wait()

        start(0, 0, sem0)
        hT_dma.wait()

        def process(par):
            sb = eb_src.at[par]
            db = eb_dst.at[par]
            wb = eb_w.at[par]

            @plsc.parallel_loop(0, _GPB, unroll=10)
            def body(g):
                o = g * 16
                src16 = sb[pl.ds(o, 16)]
                dst16 = db[pl.ds(o, 16)]
                wp = wb[pl.ds(o, 16)]
                for fi in range(_FPT):
                    vals = plsc.load_gather(hT_v.at[fi], [src16]) * wp
                    plsc.addupdate_scatter(accs[fi], [dst16], vals)

        def run(nblk):
            def blk(jj, carry):
                j0 = jj * 2
                wait(0, sem0)
                start(j0 + 1, 1, sem1)
                process(0)
                wait(1, sem1)

                @pl.when(j0 + 2 < nblk)
                def _():
                    start(j0 + 2, 0, sem0)
                process(1)
                return carry
            lax.fori_loop(0, nblk // 2, blk, 0)

        @pl.when(cid == 0)
        def _():
            run(_NBLK0)

        @pl.when(cid == 1)
        def _():
            run(_NBLK1)

        for fi in range(_FPT):
            pltpu.sync_copy(accs[fi], out_hbm.at[grp, fbase + fi])

    return k(hT_flat, ei, w)


def kernel(x, edge_index, edge_weight, W1, b1, W2, b2):
    W2p = jnp.pad(W2, ((0, 0), (0, _H - _O)))
    b1_col = b1.reshape(_H, 1)
    b2_2d = b2.reshape(1, _O)

    h1T = _mm1(x, W1)
    p1 = _sc_agg(h1T, edge_index, edge_weight)
    h2T = _mid(p1, b1_col, W2p)
    p2 = _sc_agg(h2T, edge_index, edge_weight)
    return _fin(p2, b2_2d)


# speedup vs baseline: 1.0958x; 1.0358x over previous
import functools

import jax
import jax.numpy as jnp
from jax import lax
from jax.experimental import pallas as pl
from jax.experimental.pallas import tpu as pltpu
from jax.experimental.pallas import tpu_sc as plsc

_N = 10000
_E = 320000
_F = 128
_H = 16
_O = 7

_NP = 10240
_NG = 8
_FPT = 4
_EB = 800
_NBLK0 = 66
_NBLK1 = 34
_GPB = _EB // 16
_G0 = _NBLK0 * _EB
_G1 = _NBLK1 * _EB


def _mm1_body(x_ref, w_ref, o_ref):
    hT = lax.dot_general(w_ref[:], x_ref[:], (((0,), (1,)), ((), ())),
                         preferred_element_type=jnp.float32)
    o_ref[:] = jnp.concatenate(
        [hT, jnp.zeros((_H, _NP - _N), jnp.float32)], axis=1)


def _mm1(x, W1):
    return pl.pallas_call(
        _mm1_body,
        out_shape=jax.ShapeDtypeStruct((_H, _NP), jnp.float32),
    )(x, W1)


def _mid_body(p_ref, b1_ref, w2_ref, o_ref):
    aggT = jnp.sum(p_ref[:], axis=0) + b1_ref[:]
    o_ref[:] = lax.dot_general(w2_ref[:], aggT, (((0,), (0,)), ((), ())),
                               preferred_element_type=jnp.float32)


def _mid(p1, b1_col, W2p):
    return pl.pallas_call(
        _mid_body,
        out_shape=jax.ShapeDtypeStruct((_H, _NP), jnp.float32),
    )(p1, b1_col, W2p)


def _fin_body(p_ref, b2_ref, eye_ref, o_ref):
    aggT = jnp.sum(p_ref[:], axis=0)
    agg = lax.dot_general(aggT, eye_ref[:], (((0,), (0,)), ((), ())),
                          preferred_element_type=jnp.float32)
    o_ref[:] = agg[:_N, :_O] + b2_ref[:]


def _fin(p2, b2_2d):
    return pl.pallas_call(
        _fin_body,
        out_shape=jax.ShapeDtypeStruct((_N, _O), jnp.float32),
    )(p2, b2_2d, jnp.eye(_H, dtype=jnp.float32))


def _sc_agg(hT_flat, ei, w):
    mesh = plsc.VectorSubcoreMesh(core_axis_name="c", subcore_axis_name="s")

    @functools.partial(
        pl.kernel,
        out_type=jax.ShapeDtypeStruct((_NG, _H, _NP), jnp.float32),
        mesh=mesh,
        scratch_types=[
            pltpu.VMEM((_FPT, _NP), jnp.float32),
            pltpu.VMEM((_NP,), jnp.float32),
            pltpu.VMEM((_NP,), jnp.float32),
            pltpu.VMEM((_NP,), jnp.float32),
            pltpu.VMEM((_NP,), jnp.float32),
            pltpu.VMEM((2, _EB), jnp.int32),
            pltpu.VMEM((2, _EB), jnp.int32),
            pltpu.VMEM((2, _EB), jnp.float32),
            pltpu.SemaphoreType.DMA,
            pltpu.SemaphoreType.DMA,
            pltpu.SemaphoreType.DMA,
        ],
        compiler_params=pltpu.CompilerParams(
            needs_layout_passes=False, use_tc_tiling_on_sc=False),
    )
    def k(hT_hbm, ei_hbm, w_hbm, out_hbm, hT_v, a0, a1, a2, a3,
          eb_src, eb_dst, eb_w, sem0, sem1, semh):
        accs = (a0, a1, a2, a3)
        cid = lax.axis_index("c")
        sid = lax.axis_index("s")
        wid = cid * 16 + sid
        grp = wid // _FPT
        fbase = (wid % _FPT) * _FPT

        hT_dma = pltpu.make_async_copy(
            hT_hbm.at[pl.ds(fbase, _FPT)], hT_v, semh)
        hT_dma.start()

        zero16 = jnp.zeros((16,), jnp.float32)

        @plsc.parallel_loop(0, _NP // 32, unroll=8)
        def zb(i):
            for fi in range(_FPT):
                for u in range(2):
                    accs[fi][pl.ds((i * 2 + u) * 16, 16)] = zero16

        ebase0 = jnp.where(cid == 0, grp * _G0,
                           4 * _G0 + (grp - 4) * _G1)

        def start(j, par, sem):
            e0 = pl.multiple_of(ebase0 + j * _EB, 8)
            pltpu.make_async_copy(
                ei_hbm.at[0, pl.ds(e0, _EB)], eb_src.at[par], sem).start()
            pltpu.make_async_copy(
                ei_hbm.at[1, pl.ds(e0, _EB)], eb_dst.at[par], sem).start()
            pltpu.make_async_copy(
                w_hbm.at[pl.ds(e0, _EB)], eb_w.at[par], sem).start()

        def wait(par, sem):
            pltpu.make_async_copy(
                ei_hbm.at[0, pl.ds(0, _EB)], eb_src.at[par], sem).wait()
            pltpu.make_async_copy(
                ei_hbm.at[1, pl.ds(0, _EB)], eb_dst.at[par], sem).wait()
            pltpu.make_async_copy(
                w_hbm.at[pl.ds(0, _EB)], eb_w.at[par], sem).wait()

        start(0, 0, sem0)
        hT_dma.wait()

        def process(par):
            sb = eb_src.at[par]
            db = eb_dst.at[par]
            wb = eb_w.at[par]

            @plsc.parallel_loop(0, _GPB, unroll=10)
            def body(g):
                o = g * 16
                src16 = sb[pl.ds(o, 16)]
                dst16 = db[pl.ds(o, 16)]
                wp = wb[pl.ds(o, 16)]
                for fi in range(_FPT):
                    vals = plsc.load_gather(hT_v.at[fi], [src16]) * wp
                    plsc.addupdate_scatter(accs[fi], [dst16], vals)

        def run(nblk):
            def blk(jj, carry):
                j0 = jj * 2
                wait(0, sem0)
                start(j0 + 1, 1, sem1)
                process(0)
                wait(1, sem1)

                @pl.when(j0 + 2 < nblk)
                def _():
                    start(j0 + 2, 0, sem0)
                process(1)
                return carry
            lax.fori_loop(0, nblk // 2, blk, 0)

        @pl.when(cid == 0)
        def _():
            run(_NBLK0)

        @pl.when(cid == 1)
        def _():
            run(_NBLK1)

        for fi in range(_FPT):
            pltpu.sync_copy(accs[fi], out_hbm.at[grp, fbase + fi])

    return k(hT_flat, ei, w)


def kernel(x, edge_index, edge_weight, W1, b1, W2, b2):
    W2p = jnp.pad(W2, ((0, 0), (0, _H - _O)))
    b1_col = b1.reshape(_H, 1)
    b2_2d = b2.reshape(1, _O)

    h1T = _mm1(x, W1)
    p1 = _sc_agg(h1T, edge_index, edge_weight)
    h2T = _mid(p1, b1_col, W2p)
    p2 = _sc_agg(h2T, edge_index, edge_weight)
    return _fin(p2, b2_2d)


# speedup vs baseline: 1.1319x; 1.0330x over previous
import functools

import jax
import jax.numpy as jnp
from jax import lax
from jax.experimental import pallas as pl
from jax.experimental.pallas import tpu as pltpu
from jax.experimental.pallas import tpu_sc as plsc

_N = 10000
_E = 320000
_F = 128
_H = 16
_O = 7

_NP = 10240
_NG = 8
_FPT = 4
_EB = 800
_NBLK0 = 62
_NBLK1 = 38
_GPB = _EB // 16
_G0 = _NBLK0 * _EB
_G1 = _NBLK1 * _EB


def _mm1_body(x_ref, w_ref, o_ref):
    hT = lax.dot_general(w_ref[:], x_ref[:], (((0,), (1,)), ((), ())),
                         preferred_element_type=jnp.float32)
    o_ref[:] = jnp.concatenate(
        [hT, jnp.zeros((_H, _NP - _N), jnp.float32)], axis=1)


def _mm1(x, W1):
    return pl.pallas_call(
        _mm1_body,
        out_shape=jax.ShapeDtypeStruct((_H, _NP), jnp.float32),
    )(x, W1)


def _mid_body(p_ref, b1_ref, w2_ref, o_ref):
    aggT = jnp.sum(p_ref[:], axis=0) + b1_ref[:]
    o_ref[:] = lax.dot_general(w2_ref[:], aggT, (((0,), (0,)), ((), ())),
                               preferred_element_type=jnp.float32)


def _mid(p1, b1_col, W2p):
    return pl.pallas_call(
        _mid_body,
        out_shape=jax.ShapeDtypeStruct((_H, _NP), jnp.float32),
    )(p1, b1_col, W2p)


def _fin_body(p_ref, b2_ref, eye_ref, o_ref):
    aggT = jnp.sum(p_ref[:], axis=0)
    agg = lax.dot_general(aggT, eye_ref[:], (((0,), (0,)), ((), ())),
                          preferred_element_type=jnp.float32)
    o_ref[:] = agg[:_N, :_O] + b2_ref[:]


def _fin(p2, b2_2d):
    return pl.pallas_call(
        _fin_body,
        out_shape=jax.ShapeDtypeStruct((_N, _O), jnp.float32),
    )(p2, b2_2d, jnp.eye(_H, dtype=jnp.float32))


def _sc_agg(hT_flat, ei, w):
    mesh = plsc.VectorSubcoreMesh(core_axis_name="c", subcore_axis_name="s")

    @functools.partial(
        pl.kernel,
        out_type=jax.ShapeDtypeStruct((_NG, _H, _NP), jnp.float32),
        mesh=mesh,
        scratch_types=[
            pltpu.VMEM((_FPT, _NP), jnp.float32),
            pltpu.VMEM((_NP,), jnp.float32),
            pltpu.VMEM((_NP,), jnp.float32),
            pltpu.VMEM((_NP,), jnp.float32),
            pltpu.VMEM((_NP,), jnp.float32),
            pltpu.VMEM((2, _EB), jnp.int32),
            pltpu.VMEM((2, _EB), jnp.int32),
            pltpu.VMEM((2, _EB), jnp.float32),
            pltpu.SemaphoreType.DMA,
            pltpu.SemaphoreType.DMA,
            pltpu.SemaphoreType.DMA,
        ],
        compiler_params=pltpu.CompilerParams(
            needs_layout_passes=False, use_tc_tiling_on_sc=False),
    )
    def k(hT_hbm, ei_hbm, w_hbm, out_hbm, hT_v, a0, a1, a2, a3,
          eb_src, eb_dst, eb_w, sem0, sem1, semh):
        accs = (a0, a1, a2, a3)
        cid = lax.axis_index("c")
        sid = lax.axis_index("s")
        wid = cid * 16 + sid
        grp = wid // _FPT
        fbase = (wid % _FPT) * _FPT

        hT_dma = pltpu.make_async_copy(
            hT_hbm.at[pl.ds(fbase, _FPT)], hT_v, semh)
        hT_dma.start()

        zero16 = jnp.zeros((16,), jnp.float32)

        @plsc.parallel_loop(0, _NP // 32, unroll=8)
        def zb(i):
            for fi in range(_FPT):
                for u in range(2):
                    accs[fi][pl.ds((i * 2 + u) * 16, 16)] = zero16

        ebase0 = jnp.where(cid == 0, grp * _G0,
                           4 * _G0 + (grp - 4) * _G1)

        def start(j, par, sem):
            e0 = pl.multiple_of(ebase0 + j * _EB, 8)
            pltpu.make_async_copy(
                ei_hbm.at[0, pl.ds(e0, _EB)], eb_src.at[par], sem).start()
            pltpu.make_async_copy(
                ei_hbm.at[1, pl.ds(e0, _EB)], eb_dst.at[par], sem).start()
            pltpu.make_async_copy(
                w_hbm.at[pl.ds(e0, _EB)], eb_w.at[par], sem).start()

        def wait(par, sem):
            pltpu.make_async_copy(
                ei_hbm.at[0, pl.ds(0, _EB)], eb_src.at[par], sem).wait()
            pltpu.make_async_copy(
                ei_hbm.at[1, pl.ds(0, _EB)], eb_dst.at[par], sem).wait()
            pltpu.make_async_copy(
                w_hbm.at[pl.ds(0, _EB)], eb_w.at[par], sem).wait()

        start(0, 0, sem0)
        hT_dma.wait()

        def process(par):
            sb = eb_src.at[par]
            db = eb_dst.at[par]
            wb = eb_w.at[par]

            @plsc.parallel_loop(0, _GPB, unroll=10)
            def body(g):
                o = g * 16
                src16 = sb[pl.ds(o, 16)]
                dst16 = db[pl.ds(o, 16)]
                wp = wb[pl.ds(o, 16)]
                for fi in range(_FPT):
                    vals = plsc.load_gather(hT_v.at[fi], [src16]) * wp
                    plsc.addupdate_scatter(accs[fi], [dst16], vals)

        def run(nblk):
            def blk(jj, carry):
                j0 = jj * 2
                wait(0, sem0)
                start(j0 + 1, 1, sem1)
                process(0)
                wait(1, sem1)

                @pl.when(j0 + 2 < nblk)
                def _():
                    start(j0 + 2, 0, sem0)
                process(1)
                return carry
            lax.fori_loop(0, nblk // 2, blk, 0)

        @pl.when(cid == 0)
        def _():
            run(_NBLK0)

        @pl.when(cid == 1)
        def _():
            run(_NBLK1)

        for fi in range(_FPT):
            pltpu.sync_copy(accs[fi], out_hbm.at[grp, fbase + fi])

    return k(hT_flat, ei, w)


def kernel(x, edge_index, edge_weight, W1, b1, W2, b2):
    W2p = jnp.pad(W2, ((0, 0), (0, _H - _O)))
    b1_col = b1.reshape(_H, 1)
    b2_2d = b2.reshape(1, _O)

    h1T = _mm1(x, W1)
    p1 = _sc_agg(h1T, edge_index, edge_weight)
    h2T = _mid(p1, b1_col, W2p)
    p2 = _sc_agg(h2T, edge_index, edge_weight)
    return _fin(p2, b2_2d)


# speedup vs baseline: 1.2636x; 1.1163x over previous
import functools

import jax
import jax.numpy as jnp
from jax import lax
from jax.experimental import pallas as pl
from jax.experimental.pallas import tpu as pltpu
from jax.experimental.pallas import tpu_sc as plsc

_N = 10000
_E = 320000
_F = 128
_H = 16
_O = 7

_NP = 10240
_NG = 8
_FPT = 4
_EB = 800
_NBLK0 = 50
_NBLK1 = 50
_GPB = _EB // 16
_G0 = _NBLK0 * _EB
_G1 = _NBLK1 * _EB


def _mm1_body(x_ref, w_ref, o_ref):
    hT = lax.dot_general(w_ref[:], x_ref[:], (((0,), (1,)), ((), ())),
                         preferred_element_type=jnp.float32)
    o_ref[:] = jnp.concatenate(
        [hT, jnp.zeros((_H, _NP - _N), jnp.float32)], axis=1)


def _mm1(x, W1):
    return pl.pallas_call(
        _mm1_body,
        out_shape=jax.ShapeDtypeStruct((_H, _NP), jnp.float32),
    )(x, W1)


def _mid_body(p_ref, b1_ref, w2_ref, o_ref):
    aggT = jnp.sum(p_ref[:], axis=0) + b1_ref[:]
    o_ref[:] = lax.dot_general(w2_ref[:], aggT, (((0,), (0,)), ((), ())),
                               preferred_element_type=jnp.float32)


def _mid(p1, b1_col, W2p):
    return pl.pallas_call(
        _mid_body,
        out_shape=jax.ShapeDtypeStruct((_H, _NP), jnp.float32),
    )(p1, b1_col, W2p)


def _fin_body(p_ref, b2_ref, eye_ref, o_ref):
    aggT = jnp.sum(p_ref[:], axis=0)
    agg = lax.dot_general(aggT, eye_ref[:], (((0,), (0,)), ((), ())),
                          preferred_element_type=jnp.float32)
    o_ref[:] = agg[:_N, :_O] + b2_ref[:]


def _fin(p2, b2_2d):
    return pl.pallas_call(
        _fin_body,
        out_shape=jax.ShapeDtypeStruct((_N, _O), jnp.float32),
    )(p2, b2_2d, jnp.eye(_H, dtype=jnp.float32))


def _sc_agg(hT_flat, ei, w):
    mesh = plsc.VectorSubcoreMesh(core_axis_name="c", subcore_axis_name="s")

    @functools.partial(
        pl.kernel,
        out_type=jax.ShapeDtypeStruct((_NG, _H, _NP), jnp.float32),
        mesh=mesh,
        scratch_types=[
            pltpu.VMEM((_FPT, _NP), jnp.float32),
            pltpu.VMEM((_NP,), jnp.float32),
            pltpu.VMEM((_NP,), jnp.float32),
            pltpu.VMEM((_NP,), jnp.float32),
            pltpu.VMEM((_NP,), jnp.float32),
            pltpu.VMEM((2, _EB), jnp.int32),
            pltpu.VMEM((2, _EB), jnp.int32),
            pltpu.VMEM((2, _EB), jnp.float32),
            pltpu.SemaphoreType.DMA,
            pltpu.SemaphoreType.DMA,
            pltpu.SemaphoreType.DMA,
        ],
        compiler_params=pltpu.CompilerParams(
            needs_layout_passes=False, use_tc_tiling_on_sc=False),
    )
    def k(hT_hbm, ei_hbm, w_hbm, out_hbm, hT_v, a0, a1, a2, a3,
          eb_src, eb_dst, eb_w, sem0, sem1, semh):
        accs = (a0, a1, a2, a3)
        cid = lax.axis_index("c")
        sid = lax.axis_index("s")
        wid = cid * 16 + sid
        grp = wid // _FPT
        fbase = (wid % _FPT) * _FPT

        hT_dma = pltpu.make_async_copy(
            hT_hbm.at[pl.ds(fbase, _FPT)], hT_v, semh)
        hT_dma.start()

        zero16 = jnp.zeros((16,), jnp.float32)

        @plsc.parallel_loop(0, _NP // 32, unroll=8)
        def zb(i):
            for fi in range(_FPT):
                for u in range(2):
                    accs[fi][pl.ds((i * 2 + u) * 16, 16)] = zero16

        ebase0 = jnp.where(cid == 0, grp * _G0,
                           4 * _G0 + (grp - 4) * _G1)

        def start(j, par, sem):
            e0 = pl.multiple_of(ebase0 + j * _EB, 8)
            pltpu.make_async_copy(
                ei_hbm.at[0, pl.ds(e0, _EB)], eb_src.at[par], sem).start()
            pltpu.make_async_copy(
                ei_hbm.at[1, pl.ds(e0, _EB)], eb_dst.at[par], sem).start()
            pltpu.make_async_copy(
                w_hbm.at[pl.ds(e0, _EB)], eb_w.at[par], sem).start()

        def wait(par, sem):
            pltpu.make_async_copy(
                ei_hbm.at[0, pl.ds(0, _EB)], eb_src.at[par], sem).wait()
            pltpu.make_async_copy(
                ei_hbm.at[1, pl.ds(0, _EB)], eb_dst.at[par], sem).wait()
            pltpu.make_async_copy(
                w_hbm.at[pl.ds(0, _EB)], eb_w.at[par], sem).wait()

        start(0, 0, sem0)
        hT_dma.wait()

        def process(par):
            sb = eb_src.at[par]
            db = eb_dst.at[par]
            wb = eb_w.at[par]

            @plsc.parallel_loop(0, _GPB, unroll=10)
            def body(g):
                o = g * 16
                src16 = sb[pl.ds(o, 16)]
                dst16 = db[pl.ds(o, 16)]
                wp = wb[pl.ds(o, 16)]
                for fi in range(_FPT):
                    vals = plsc.load_gather(hT_v.at[fi], [src16]) * wp
                    plsc.addupdate_scatter(accs[fi], [dst16], vals)

        def run(nblk):
            def blk(jj, carry):
                j0 = jj * 2
                wait(0, sem0)
                start(j0 + 1, 1, sem1)
                process(0)
                wait(1, sem1)

                @pl.when(j0 + 2 < nblk)
                def _():
                    start(j0 + 2, 0, sem0)
                process(1)
                return carry
            lax.fori_loop(0, nblk // 2, blk, 0)

        @pl.when(cid == 0)
        def _():
            run(_NBLK0)

        @pl.when(cid == 1)
        def _():
            run(_NBLK1)

        for fi in range(_FPT):
            pltpu.sync_copy(accs[fi], out_hbm.at[grp, fbase + fi])

    return k(hT_flat, ei, w)


def kernel(x, edge_index, edge_weight, W1, b1, W2, b2):
    W2p = jnp.pad(W2, ((0, 0), (0, _H - _O)))
    b1_col = b1.reshape(_H, 1)
    b2_2d = b2.reshape(1, _O)

    h1T = _mm1(x, W1)
    p1 = _sc_agg(h1T, edge_index, edge_weight)
    h2T = _mid(p1, b1_col, W2p)
    p2 = _sc_agg(h2T, edge_index, edge_weight)
    return _fin(p2, b2_2d)
